# baseline (speedup 1.0000x reference)
import functools
import jax
import jax.numpy as jnp
from jax import lax
from jax.experimental import pallas as pl
from jax.experimental.pallas import tpu as pltpu
from jax.experimental.pallas import tpu_sc as plsc


def kernel(x, emb_table):
    seq, batch, d = x.shape
    d_emb = emb_table.shape[1]
    d_out = d + d_emb

    nc, ns = 2, 16
    nw = nc * ns
    rows_per_w = seq // nw
    r = 32
    chunks = rows_per_w // r

    mesh = plsc.VectorSubcoreMesh(
        core_axis_name="c", subcore_axis_name="s", num_cores=nc, num_subcores=ns
    )

    @functools.partial(
        pl.kernel,
        out_type=jax.ShapeDtypeStruct((seq, batch, d_out), jnp.float32),
        mesh=mesh,
        scratch_types=[
            pltpu.VMEM((r, d_emb), jnp.float32),
            pltpu.VMEM((r, d_emb), jnp.float32),
            pltpu.SemaphoreType.DMA,
            pltpu.SemaphoreType.DMA,
            pltpu.SemaphoreType.DMA,
            pltpu.SemaphoreType.DMA,
        ],
    )
    def sc_k(x_hbm, emb_hbm, out_hbm, eb0, eb1, rsem0, rsem1, wsem, xsem):
        wid = lax.axis_index("s") * nc + lax.axis_index("c")
        base = wid * rows_per_w
        bufs = (eb0, eb1)
        rsems = (rsem0, rsem1)

        x_copy = pltpu.async_copy(
            x_hbm.at[pl.ds(base, rows_per_w)],
            out_hbm.at[pl.ds(base, rows_per_w), :, pl.ds(0, d)],
            xsem,
        )

        reads = [None] * chunks
        writes = [None] * chunks
        reads[0] = pltpu.async_copy(
            emb_hbm.at[pl.ds(base, r), :], bufs[0], rsems[0]
        )
        for ci in range(chunks):
            if ci >= 1:
                for h in writes[ci - 1]:
                    h.wait()
            if ci + 1 < chunks:
                reads[ci + 1] = pltpu.async_copy(
                    emb_hbm.at[pl.ds(base + (ci + 1) * r, r), :],
                    bufs[(ci + 1) % 2],
                    rsems[(ci + 1) % 2],
                )
            reads[ci].wait()
            r0 = base + ci * r
            writes[ci] = [
                pltpu.async_copy(
                    bufs[ci % 2],
                    out_hbm.at[pl.ds(r0, r), b, pl.ds(d, d_emb)],
                    wsem,
                )
                for b in range(batch)
            ]
        for h in writes[chunks - 1]:
            h.wait()
        x_copy.wait()

    return sc_k(x, emb_table)


# speedup vs baseline: 22.2624x; 22.2624x over previous
import functools
import jax
import jax.numpy as jnp
from jax import lax
from jax.experimental import pallas as pl
from jax.experimental.pallas import tpu as pltpu
from jax.experimental.pallas import tpu_sc as plsc


def _x_body(out_alias_ref, x_ref, out_ref):
    out_ref[...] = x_ref[...]


def kernel(x, emb_table):
    seq, batch, d = x.shape
    d_emb = emb_table.shape[1]
    d_out = d + d_emb

    nc, ns = 2, 16
    nw = nc * ns
    rows_per_w = seq // nw
    r = 16
    chunks = rows_per_w // r
    nbuf = 4

    mesh = plsc.VectorSubcoreMesh(
        core_axis_name="c", subcore_axis_name="s", num_cores=nc, num_subcores=ns
    )

    @functools.partial(
        pl.kernel,
        out_type=jax.ShapeDtypeStruct((seq, batch, d_out), jnp.float32),
        mesh=mesh,
        scratch_types=(
            [pltpu.VMEM((r, d_emb), jnp.float32) for _ in range(nbuf)]
            + [pltpu.SemaphoreType.DMA for _ in range(nbuf)]
            + [pltpu.SemaphoreType.DMA]
        ),
    )
    def sc_fill(emb_hbm, out_hbm, *scratch):
        bufs = scratch[:nbuf]
        rsems = scratch[nbuf:2 * nbuf]
        wsem = scratch[2 * nbuf]
        wid = lax.axis_index("s") * nc + lax.axis_index("c")
        base = wid * rows_per_w

        def read(k):
            return pltpu.async_copy(
                emb_hbm.at[pl.ds(base + k * r, r), :],
                bufs[k % nbuf],
                rsems[k % nbuf],
            )

        reads = [None] * chunks
        writes = [None] * chunks
        for k in range(min(nbuf - 1, chunks)):
            reads[k] = read(k)
        for ci in range(chunks):
            nxt = ci + nbuf - 1
            if nxt < chunks:
                prev = nxt - nbuf
                if prev >= 0:
                    for h in writes[prev]:
                        h.wait()
                    writes[prev] = None
                reads[nxt] = read(nxt)
            reads[ci].wait()
            r0 = base + ci * r
            writes[ci] = [
                pltpu.async_copy(
                    bufs[ci % nbuf],
                    out_hbm.at[pl.ds(r0, r), b, pl.ds(d, d_emb)],
                    wsem,
                )
                for b in range(batch)
            ]
        for ci in range(chunks):
            if writes[ci] is not None:
                for h in writes[ci]:
                    h.wait()

    out1 = sc_fill(emb_table)

    bs = 512
    grid = (seq // bs,)
    return pl.pallas_call(
        _x_body,
        grid=grid,
        in_specs=[
            pl.BlockSpec(memory_space=pl.ANY),
            pl.BlockSpec((bs, batch, d), lambda i: (i, 0, 0)),
        ],
        out_specs=pl.BlockSpec((bs, batch, d), lambda i: (i, 0, 0)),
        out_shape=jax.ShapeDtypeStruct((seq, batch, d_out), x.dtype),
        input_output_aliases={0: 0},
    )(out1, x)


# speedup vs baseline: 22.6935x; 1.0194x over previous
import functools
import jax
import jax.numpy as jnp
from jax import lax
from jax.experimental import pallas as pl
from jax.experimental.pallas import tpu as pltpu
from jax.experimental.pallas import tpu_sc as plsc


def _x_body(out_alias_ref, x_ref, out_ref):
    out_ref[...] = x_ref[...]


def kernel(x, emb_table):
    seq, batch, d = x.shape
    d_emb = emb_table.shape[1]
    d_out = d + d_emb

    nc, ns = 2, 16
    nw = nc * ns
    rows_per_w = seq // nw
    r = 32
    chunks = rows_per_w // r

    mesh = plsc.VectorSubcoreMesh(
        core_axis_name="c", subcore_axis_name="s", num_cores=nc, num_subcores=ns
    )

    @functools.partial(
        pl.kernel,
        out_type=jax.ShapeDtypeStruct((seq, batch, d_out), jnp.float32),
        mesh=mesh,
        scratch_types=[
            pltpu.VMEM((r, d_emb), jnp.float32),
            pltpu.VMEM((r, d_emb), jnp.float32),
            pltpu.SemaphoreType.DMA,
            pltpu.SemaphoreType.DMA,
            pltpu.SemaphoreType.DMA,
        ],
    )
    def sc_fill(emb_hbm, out_hbm, eb0, eb1, rsem0, rsem1, wsem):
        wid = lax.axis_index("s") * nc + lax.axis_index("c")
        base = wid * rows_per_w
        bufs = (eb0, eb1)
        rsems = (rsem0, rsem1)

        reads = [None] * chunks
        writes = [None] * chunks
        reads[0] = pltpu.async_copy(
            emb_hbm.at[pl.ds(base, r), :], bufs[0], rsems[0]
        )
        for ci in range(chunks):
            if ci >= 1:
                for h in writes[ci - 1]:
                    h.wait()
            if ci + 1 < chunks:
                reads[ci + 1] = pltpu.async_copy(
                    emb_hbm.at[pl.ds(base + (ci + 1) * r, r), :],
                    bufs[(ci + 1) % 2],
                    rsems[(ci + 1) % 2],
                )
            reads[ci].wait()
            r0 = base + ci * r
            writes[ci] = [
                pltpu.async_copy(
                    bufs[ci % 2],
                    out_hbm.at[pl.ds(r0, r), b, pl.ds(d, d_emb)],
                    wsem,
                )
                for b in range(batch)
            ]
        for h in writes[chunks - 1]:
            h.wait()

    out1 = sc_fill(emb_table)

    bs = 512
    grid = (seq // bs,)
    return pl.pallas_call(
        _x_body,
        grid=grid,
        in_specs=[
            pl.BlockSpec(memory_space=pl.ANY),
            pl.BlockSpec((bs, batch, d), lambda i: (i, 0, 0)),
        ],
        out_specs=pl.BlockSpec((bs, batch, d), lambda i: (i, 0, 0)),
        out_shape=jax.ShapeDtypeStruct((seq, batch, d_out), x.dtype),
        input_output_aliases={0: 0},
    )(out1, x)
